# baseline (speedup 1.0000x reference)
import functools

import jax
import jax.numpy as jnp
from jax import lax
from jax.experimental import pallas as pl
from jax.experimental.pallas import tpu as pltpu
from jax.experimental.pallas import tpu_sc as plsc

_GRP = 16


def _embedding_lookup(tokens_flat, table):
    B, = tokens_flat.shape
    V, D = table.shape
    info = plsc.get_sparse_core_info()
    NC, NS = info.num_cores, info.num_subcores
    NW = NC * NS
    b_per_w = B // NW
    n_grp = b_per_w // _GRP
    assert B == NW * b_per_w and b_per_w == n_grp * _GRP

    mesh = plsc.VectorSubcoreMesh(core_axis_name="c", subcore_axis_name="s")

    @functools.partial(
        pl.kernel,
        mesh=mesh,
        out_type=jax.ShapeDtypeStruct((B, D), jnp.float32),
        scratch_types=[
            pltpu.VMEM((b_per_w,), jnp.int32),
            pltpu.VMEM((_GRP,), jnp.int32),
            pltpu.SemaphoreType.DMA,
            pltpu.SemaphoreType.DMA,
        ],
    )
    def gather_kernel(idx_hbm, table_hbm, out_hbm, idx_v, tbuf, sem0, sem1):
        sems = (sem0, sem1)
        wid = lax.axis_index("s") * NC + lax.axis_index("c")
        base = wid * b_per_w
        pltpu.sync_copy(idx_hbm.at[pl.ds(base, b_per_w)], idx_v)

        def issue_group(g, sem):
            i0 = g * _GRP
            pltpu.sync_copy(idx_hbm.at[pl.ds(base + i0, _GRP)], tbuf)
            tvec = tbuf[...]
            for j in range(_GRP):
                t = tvec[j]
                pltpu.async_copy(table_hbm.at[pl.ds(t, 1)],
                                 out_hbm.at[pl.ds(base + i0 + j, 1)], sem)

        def wait_group(sem):
            def w(j, carry):
                pltpu.make_async_copy(table_hbm.at[pl.ds(0, 1)],
                                      out_hbm.at[pl.ds(base, 1)], sem).wait()
                return carry
            lax.fori_loop(0, _GRP, w, 0)

        issue_group(0, sems[0])

        def body(g, carry):
            for p in range(2):
                gg = 1 + 2 * g + p
                issue_group(gg, sems[(1 + p) % 2])
                wait_group(sems[p % 2])
            return carry

        lax.fori_loop(0, (n_grp - 1) // 2, body, 0)

        issue_group(n_grp - 1, sems[(n_grp - 1) % 2])
        wait_group(sems[n_grp % 2])
        wait_group(sems[(n_grp - 1) % 2])

    return gather_kernel(tokens_flat, table)


def kernel(tokens, start_pos, tok_embeddings_weight):
    B, S = tokens.shape
    V, D = tok_embeddings_weight.shape
    out = _embedding_lookup(tokens.reshape(B * S), tok_embeddings_weight)
    return out.reshape(B, S, D)


# speedup vs baseline: 36.1268x; 36.1268x over previous
import functools

import jax
import jax.numpy as jnp
from jax import lax
from jax.experimental import pallas as pl
from jax.experimental.pallas import tpu as pltpu
from jax.experimental.pallas import tpu_sc as plsc

_CHUNK = 16
_NBUF = 4


def _embedding_lookup(tokens_flat, table):
    B, = tokens_flat.shape
    V, D = table.shape
    info = plsc.get_sparse_core_info()
    NC, NS = info.num_cores, info.num_subcores
    NW = NC * NS
    b_per_w = B // NW
    n_chunks = b_per_w // _CHUNK
    assert B == NW * b_per_w and b_per_w == n_chunks * _CHUNK

    idx2d = tokens_flat.reshape(B // _CHUNK, _CHUNK)
    mesh = plsc.VectorSubcoreMesh(core_axis_name="c", subcore_axis_name="s")

    @functools.partial(
        pl.kernel,
        mesh=mesh,
        out_type=jax.ShapeDtypeStruct((B, D), jnp.float32),
        scratch_types=[
            pltpu.VMEM((n_chunks, _CHUNK), jnp.int32),
        ]
        + [pltpu.VMEM((_CHUNK, D), jnp.float32)] * _NBUF
        + [pltpu.SemaphoreType.DMA] * (2 * _NBUF),
    )
    def gather_kernel(idx_hbm, table_hbm, out_hbm, idx_v, *bufs_sems):
        bufs = bufs_sems[:_NBUF]
        gsems = bufs_sems[_NBUF:2 * _NBUF]
        ssems = bufs_sems[2 * _NBUF:]
        wid = lax.axis_index("s") * NC + lax.axis_index("c")
        base_chunk = wid * n_chunks
        pltpu.sync_copy(idx_hbm.at[pl.ds(base_chunk, n_chunks)], idx_v)

        def out_slice(i):
            return out_hbm.at[pl.ds((base_chunk + i) * _CHUNK, _CHUNK)]

        def start_gather(i, b):
            pltpu.async_copy(table_hbm.at[idx_v.at[i]], bufs[b], gsems[b])

        for b in range(_NBUF - 1):
            start_gather(b, b)

        def step(i, b, first, last):
            pltpu.make_async_copy(table_hbm.at[idx_v.at[0]], bufs[b],
                                  gsems[b]).wait()
            pltpu.async_copy(bufs[b], out_slice(i), ssems[b])
            if not last:
                nb = (b + _NBUF - 1) % _NBUF
                if not first:
                    pltpu.make_async_copy(bufs[nb], out_slice(0),
                                          ssems[nb]).wait()
                start_gather(i + _NBUF - 1, nb)

        step(0, 0, first=True, last=False)

        def body(grp, carry):
            for k in range(_NBUF):
                i = 1 + _NBUF * grp + k
                step(i, (1 + k) % _NBUF, first=False, last=False)
            return carry

        n_steady = (n_chunks - 1 - (_NBUF - 1)) // _NBUF
        lax.fori_loop(0, n_steady, body, 0)

        for k in range(_NBUF - 1):
            i = n_chunks - (_NBUF - 1) + k
            step(i, i % _NBUF, first=False, last=True)

        for b in range(_NBUF):
            pltpu.make_async_copy(bufs[b], out_slice(0), ssems[b]).wait()

    return gather_kernel(idx2d, table)


def kernel(tokens, start_pos, tok_embeddings_weight):
    B, S = tokens.shape
    V, D = tok_embeddings_weight.shape
    out = _embedding_lookup(tokens.reshape(B * S), tok_embeddings_weight)
    return out.reshape(B, S, D)


# speedup vs baseline: 36.2897x; 1.0045x over previous
import functools

import jax
import jax.numpy as jnp
from jax import lax
from jax.experimental import pallas as pl
from jax.experimental.pallas import tpu as pltpu
from jax.experimental.pallas import tpu_sc as plsc

_CHUNK = 16
_NBUF = 4


def _embedding_lookup(tokens_flat, table):
    B, = tokens_flat.shape
    V, D = table.shape
    info = plsc.get_sparse_core_info()
    NC, NS = info.num_cores, info.num_subcores
    NW = NC * NS
    b_per_w = B // NW
    n_chunks = b_per_w // _CHUNK
    assert B == NW * b_per_w and b_per_w == n_chunks * _CHUNK

    mesh = plsc.VectorSubcoreMesh(core_axis_name="c", subcore_axis_name="s")

    @functools.partial(
        pl.kernel,
        mesh=mesh,
        out_type=jax.ShapeDtypeStruct((B, D), jnp.float32),
        scratch_types=[
            pltpu.VMEM((b_per_w,), jnp.int32),
        ]
        + [pltpu.VMEM((_CHUNK, D), jnp.float32)] * _NBUF
        + [pltpu.SemaphoreType.DMA] * (2 * _NBUF),
    )
    def gather_kernel(idx_hbm, table_hbm, out_hbm, idx_v, *bufs_sems):
        bufs = bufs_sems[:_NBUF]
        gsems = bufs_sems[_NBUF:2 * _NBUF]
        ssems = bufs_sems[2 * _NBUF:]
        wid = lax.axis_index("s") * NC + lax.axis_index("c")
        base = wid * b_per_w
        pltpu.sync_copy(idx_hbm.at[pl.ds(base, b_per_w)], idx_v)

        def out_slice(i):
            return out_hbm.at[pl.ds(base + i * _CHUNK, _CHUNK)]

        def start_gather(i, b):
            off = pl.multiple_of(i * _CHUNK, _CHUNK)
            pltpu.async_copy(table_hbm.at[idx_v.at[pl.ds(off, _CHUNK)]],
                             bufs[b], gsems[b])

        for b in range(_NBUF - 1):
            start_gather(b, b)

        def step(i, b, first=False, last=False):
            pltpu.make_async_copy(table_hbm.at[idx_v.at[pl.ds(0, _CHUNK)]],
                                  bufs[b], gsems[b]).wait()
            pltpu.async_copy(bufs[b], out_slice(i), ssems[b])
            if not last:
                nb = (b + _NBUF - 1) % _NBUF
                if not first:
                    pltpu.make_async_copy(bufs[nb], out_slice(0),
                                          ssems[nb]).wait()
                start_gather(i + _NBUF - 1, nb)

        step(0, 0, first=True)

        n_steady = (n_chunks - _NBUF - ((n_chunks - 1) % _NBUF)) // _NBUF

        def body(grp, carry):
            for k in range(_NBUF):
                i = 1 + _NBUF * grp + k
                step(i, (1 + k) % _NBUF)
            return carry

        lax.fori_loop(0, n_steady, body, 0)

        for i in range(1 + n_steady * _NBUF, n_chunks - (_NBUF - 1)):
            step(i, i % _NBUF)
        for i in range(n_chunks - (_NBUF - 1), n_chunks):
            step(i, i % _NBUF, last=True)

        for b in range(_NBUF):
            pltpu.make_async_copy(bufs[b], out_slice(0), ssems[b]).wait()

    return gather_kernel(tokens_flat, table)


def kernel(tokens, start_pos, tok_embeddings_weight):
    B, S = tokens.shape
    V, D = tok_embeddings_weight.shape
    out = _embedding_lookup(tokens.reshape(B * S), tok_embeddings_weight)
    return out.reshape(B, S, D)
